# Initial kernel scaffold; baseline (speedup 1.0000x reference)
import jax
import jax.numpy as jnp
from jax.experimental import pallas as pl


def kernel(x_author, x_paper, edge_index_writes, edge_index_rev, params):
    raise NotImplementedError("write your pallas kernel here")


# speedup vs baseline: 3.8319x; 3.8319x over previous
import functools

import jax
import jax.numpy as jnp
from jax import lax
from jax.experimental import pallas as pl
from jax.experimental.pallas import tpu as pltpu
from jax.experimental.pallas import tpu_sc as plsc

N = 10000
E = 320000
D = 128

_NTILES = 16
_EPT = E // _NTILES
_CH = 80
_NCH = _EPT // _CH
_RPT = 624
_RPT_LAST = N - _RPT * (_NTILES - 1)

_BR = 1000


def _row_shard(sid, do):
    @pl.when(sid < _NTILES - 1)
    def _():
        do(sid * _RPT, _RPT)

    @pl.when(sid == _NTILES - 1)
    def _():
        do((_NTILES - 1) * _RPT, _RPT_LAST)


def _edge_loop(x_hbm, src_hbm, dst_hbm, sidx, didx, rows, acc_sh, sem, sid):
    def step(g, carry):
        base = sid * _EPT + g * _CH
        pltpu.sync_copy(src_hbm.at[pl.ds(base, _CH)], sidx)
        pltpu.async_copy(x_hbm.at[sidx], rows, sem).wait()
        pltpu.sync_copy(dst_hbm.at[pl.ds(base, _CH)], didx)
        pltpu.sync_copy(rows, acc_sh.at[didx], add=True)
        return carry

    lax.fori_loop(0, _NCH, step, 0)


@functools.partial(
    pl.kernel,
    mesh=plsc.VectorSubcoreMesh(core_axis_name="c", subcore_axis_name="s"),
    out_type=jax.ShapeDtypeStruct((2, N, D), jnp.float32),
    scratch_types=[
        pltpu.VMEM((_CH,), jnp.int32),
        pltpu.VMEM((_CH,), jnp.int32),
        pltpu.VMEM((_CH, D), jnp.float32),
        pltpu.VMEM_SHARED((N, D), jnp.float32),
        pltpu.SemaphoreType.DMA,
    ],
)
def _sc_agg(xa, xp, srcw, dstw, srcr, dstr, zd,
            out_agg, sidx, didx, rows, acc_sh, sem):
    cid = lax.axis_index("c")
    sid = lax.axis_index("s")
    _row_shard(sid, lambda r0, nr: pltpu.sync_copy(
        zd.at[pl.ds(r0, nr)], acc_sh.at[pl.ds(r0, nr)]))
    plsc.subcore_barrier()

    @pl.when(cid == 0)
    def _():
        _edge_loop(xa, srcw, dstw, sidx, didx, rows, acc_sh, sem, sid)

    @pl.when(cid == 1)
    def _():
        _edge_loop(xp, srcr, dstr, sidx, didx, rows, acc_sh, sem, sid)

    plsc.subcore_barrier()
    _row_shard(sid, lambda r0, nr: pltpu.sync_copy(
        acc_sh.at[pl.ds(r0, nr)], out_agg.at[cid, pl.ds(r0, nr)]))


@functools.partial(
    pl.kernel,
    mesh=plsc.VectorSubcoreMesh(core_axis_name="c", subcore_axis_name="s"),
    out_type=jax.ShapeDtypeStruct((2, N, D), jnp.float32),
    scratch_types=[
        pltpu.VMEM((_CH,), jnp.int32),
        pltpu.VMEM((_CH, D), jnp.float32),
        pltpu.VMEM_SHARED((N, D), jnp.float32),
    ],
)
def _sc_counts(dstw, dstr, zd, ones_h, out_cnt, didx, ones_v, cnt_sh):
    cid = lax.axis_index("c")
    sid = lax.axis_index("s")
    _row_shard(sid, lambda r0, nr: pltpu.sync_copy(
        zd.at[pl.ds(r0, nr)], cnt_sh.at[pl.ds(r0, nr)]))
    pltpu.sync_copy(ones_h, ones_v)
    plsc.subcore_barrier()

    def count_loop(dst_hbm):
        def step(g, carry):
            base = sid * _EPT + g * _CH
            pltpu.sync_copy(dst_hbm.at[pl.ds(base, _CH)], didx)
            pltpu.sync_copy(ones_v, cnt_sh.at[didx], add=True)
            return carry

        lax.fori_loop(0, _NCH, step, 0)

    @pl.when(cid == 0)
    def _():
        count_loop(dstw)

    @pl.when(cid == 1)
    def _():
        count_loop(dstr)

    plsc.subcore_barrier()
    _row_shard(sid, lambda r0, nr: pltpu.sync_copy(
        cnt_sh.at[pl.ds(r0, nr)], out_cnt.at[cid, pl.ds(r0, nr)]))


def _mmT(a, b):
    return lax.dot_general(a, b, (((1,), (1,)), ((), ())),
                           preferred_element_type=jnp.float32)


def _combine_body(aggw, cntw, aggr, cntr, xa, xp,
                  wlw, wrw, bw, wlr, wrr, br, oa, op):
    mw = aggw[...] / jnp.maximum(cntw[...][:, 0:1], 1.0)
    mr = aggr[...] / jnp.maximum(cntr[...][:, 0:1], 1.0)
    op[...] = _mmT(mw, wlw[...]) + _mmT(xp[...], wrw[...]) + bw[...]
    oa[...] = _mmT(mr, wlr[...]) + _mmT(xa[...], wrr[...]) + br[...]


def _combine(aggw, cntw, aggr, cntr, xa, xp, lp):
    row_spec = pl.BlockSpec((_BR, D), lambda i: (i, 0))
    w_spec = pl.BlockSpec((D, D), lambda i: (0, 0))
    b_spec = pl.BlockSpec((1, D), lambda i: (0, 0))
    oa, op = pl.pallas_call(
        _combine_body,
        grid=(N // _BR,),
        in_specs=[row_spec, row_spec, row_spec, row_spec, row_spec, row_spec,
                  w_spec, w_spec, b_spec, w_spec, w_spec, b_spec],
        out_specs=[row_spec, row_spec],
        out_shape=[
            jax.ShapeDtypeStruct((N, D), jnp.float32),
            jax.ShapeDtypeStruct((N, D), jnp.float32),
        ],
    )(aggw, cntw, aggr, cntr, xa, xp,
      lp["writes"]["Wl"], lp["writes"]["Wr"], lp["writes"]["b"].reshape(1, D),
      lp["rev"]["Wl"], lp["rev"]["Wr"], lp["rev"]["b"].reshape(1, D))
    return oa, op


def kernel(x_author, x_paper, edge_index_writes, edge_index_rev, params):
    xa, xp = x_author, x_paper
    srcw, dstw = edge_index_writes[0], edge_index_writes[1]
    srcr, dstr = edge_index_rev[0], edge_index_rev[1]
    zd = jnp.zeros((N, D), jnp.float32)
    ones_h = jnp.ones((_CH, D), jnp.float32)
    cnt = _sc_counts(dstw, dstr, zd, ones_h)
    for i in range(3):
        lp = params["l%d" % i]
        agg = _sc_agg(xa, xp, srcw, dstw, srcr, dstr, zd)
        xa, xp = _combine(agg[0], cnt[0], agg[1], cnt[1], xa, xp, lp)
    return (xa, xp)


# speedup vs baseline: 7.1779x; 1.8732x over previous
import functools

import jax
import jax.numpy as jnp
from jax import lax
from jax.experimental import pallas as pl
from jax.experimental.pallas import tpu as pltpu
from jax.experimental.pallas import tpu_sc as plsc

N = 10000
E = 320000
D = 128

_NTILES = 16
_CH = 128
_NCHT = 157
_EPT = _NCHT * _CH
_EPAD = _NTILES * _EPT
_NP = N + 8
_RPT = 624
_RPT_LAST_W = N - _RPT * (_NTILES - 1)
_RPT_LAST_Z = _NP - _RPT * (_NTILES - 1)

_BR = 1000


def _row_shard(sid, last, do):
    @pl.when(sid < _NTILES - 1)
    def _():
        do(sid * _RPT, _RPT)

    @pl.when(sid == _NTILES - 1)
    def _():
        do((_NTILES - 1) * _RPT, last)


def _edge_pipe(x_hbm, src_hbm, dst_hbm, base,
               sidx0, sidx1, didx0, didx1, rows0, rows1, acc_sh, sem0, sem1):
    pltpu.sync_copy(src_hbm.at[pl.ds(base, _CH)], sidx0)
    pltpu.async_copy(x_hbm.at[sidx0], rows0, sem0)

    def step(gg, carry):
        b0 = base + 2 * gg * _CH
        pltpu.sync_copy(src_hbm.at[pl.ds(b0 + _CH, _CH)], sidx1)
        pltpu.async_copy(x_hbm.at[sidx1], rows1, sem1)
        pltpu.sync_copy(dst_hbm.at[pl.ds(b0, _CH)], didx0)
        pltpu.make_async_copy(x_hbm.at[sidx0], rows0, sem0).wait()
        pltpu.sync_copy(rows0, acc_sh.at[didx0], add=True)
        pltpu.sync_copy(src_hbm.at[pl.ds(b0 + 2 * _CH, _CH)], sidx0)
        pltpu.async_copy(x_hbm.at[sidx0], rows0, sem0)
        pltpu.sync_copy(dst_hbm.at[pl.ds(b0 + _CH, _CH)], didx1)
        pltpu.make_async_copy(x_hbm.at[sidx1], rows1, sem1).wait()
        pltpu.sync_copy(rows1, acc_sh.at[didx1], add=True)
        return carry

    lax.fori_loop(0, (_NCHT - 1) // 2, step, 0)
    blast = base + (_NCHT - 1) * _CH
    pltpu.sync_copy(dst_hbm.at[pl.ds(blast, _CH)], didx0)
    pltpu.make_async_copy(x_hbm.at[sidx0], rows0, sem0).wait()
    pltpu.sync_copy(rows0, acc_sh.at[didx0], add=True)


@functools.partial(
    pl.kernel,
    mesh=plsc.VectorSubcoreMesh(core_axis_name="c", subcore_axis_name="s"),
    out_type=jax.ShapeDtypeStruct((2, N, D), jnp.float32),
    scratch_types=[
        pltpu.VMEM((_CH,), jnp.int32),
        pltpu.VMEM((_CH,), jnp.int32),
        pltpu.VMEM((_CH,), jnp.int32),
        pltpu.VMEM((_CH,), jnp.int32),
        pltpu.VMEM((_CH, D), jnp.float32),
        pltpu.VMEM((_CH, D), jnp.float32),
        pltpu.VMEM_SHARED((_NP, D), jnp.float32),
        pltpu.SemaphoreType.DMA,
        pltpu.SemaphoreType.DMA,
    ],
)
def _sc_agg(xa, xp, srcw, dstw, srcr, dstr, zd, out_agg,
            sidx0, sidx1, didx0, didx1, rows0, rows1, acc_sh, sem0, sem1):
    cid = lax.axis_index("c")
    sid = lax.axis_index("s")
    _row_shard(sid, _RPT_LAST_Z, lambda r0, nr: pltpu.sync_copy(
        zd.at[pl.ds(r0, nr)], acc_sh.at[pl.ds(r0, nr)]))
    plsc.subcore_barrier()
    base = sid * _EPT

    @pl.when(cid == 0)
    def _():
        _edge_pipe(xa, srcw, dstw, base, sidx0, sidx1, didx0, didx1,
                   rows0, rows1, acc_sh, sem0, sem1)

    @pl.when(cid == 1)
    def _():
        _edge_pipe(xp, srcr, dstr, base, sidx0, sidx1, didx0, didx1,
                   rows0, rows1, acc_sh, sem0, sem1)

    plsc.subcore_barrier()
    _row_shard(sid, _RPT_LAST_W, lambda r0, nr: pltpu.sync_copy(
        acc_sh.at[pl.ds(r0, nr)], out_agg.at[cid, pl.ds(r0, nr)]))


@functools.partial(
    pl.kernel,
    mesh=plsc.VectorSubcoreMesh(core_axis_name="c", subcore_axis_name="s"),
    out_type=jax.ShapeDtypeStruct((2, N, D), jnp.float32),
    scratch_types=[
        pltpu.VMEM((_CH,), jnp.int32),
        pltpu.VMEM((_CH, D), jnp.float32),
        pltpu.VMEM_SHARED((_NP, D), jnp.float32),
    ],
)
def _sc_counts(dstw, dstr, zd, ones_h, out_cnt, didx, ones_v, cnt_sh):
    cid = lax.axis_index("c")
    sid = lax.axis_index("s")
    _row_shard(sid, _RPT_LAST_Z, lambda r0, nr: pltpu.sync_copy(
        zd.at[pl.ds(r0, nr)], cnt_sh.at[pl.ds(r0, nr)]))
    pltpu.sync_copy(ones_h, ones_v)
    plsc.subcore_barrier()
    base = sid * _EPT

    def count_loop(dst_hbm):
        def step(g, carry):
            pltpu.sync_copy(dst_hbm.at[pl.ds(base + g * _CH, _CH)], didx)
            pltpu.sync_copy(ones_v, cnt_sh.at[didx], add=True)
            return carry

        lax.fori_loop(0, _NCHT, step, 0)

    @pl.when(cid == 0)
    def _():
        count_loop(dstw)

    @pl.when(cid == 1)
    def _():
        count_loop(dstr)

    plsc.subcore_barrier()
    _row_shard(sid, _RPT_LAST_W, lambda r0, nr: pltpu.sync_copy(
        cnt_sh.at[pl.ds(r0, nr)], out_cnt.at[cid, pl.ds(r0, nr)]))


def _pad_edges(a):
    pad = (jnp.arange(_EPAD - E, dtype=jnp.int32) % 8) + N
    return jnp.concatenate([a, pad])


def _mmT(a, b):
    return lax.dot_general(a, b, (((1,), (1,)), ((), ())),
                           preferred_element_type=jnp.float32)


def _combine_body(aggw, cntw, aggr, cntr, xa, xp,
                  wlw, wrw, bw, wlr, wrr, br, oa, op):
    mw = aggw[...] / jnp.maximum(cntw[...][:, 0:1], 1.0)
    mr = aggr[...] / jnp.maximum(cntr[...][:, 0:1], 1.0)
    op[...] = _mmT(mw, wlw[...]) + _mmT(xp[...], wrw[...]) + bw[...]
    oa[...] = _mmT(mr, wlr[...]) + _mmT(xa[...], wrr[...]) + br[...]


def _combine(aggw, cntw, aggr, cntr, xa, xp, lp):
    row_spec = pl.BlockSpec((_BR, D), lambda i: (i, 0))
    w_spec = pl.BlockSpec((D, D), lambda i: (0, 0))
    b_spec = pl.BlockSpec((1, D), lambda i: (0, 0))
    oa, op = pl.pallas_call(
        _combine_body,
        grid=(N // _BR,),
        in_specs=[row_spec, row_spec, row_spec, row_spec, row_spec, row_spec,
                  w_spec, w_spec, b_spec, w_spec, w_spec, b_spec],
        out_specs=[row_spec, row_spec],
        out_shape=[
            jax.ShapeDtypeStruct((N, D), jnp.float32),
            jax.ShapeDtypeStruct((N, D), jnp.float32),
        ],
    )(aggw, cntw, aggr, cntr, xa, xp,
      lp["writes"]["Wl"], lp["writes"]["Wr"], lp["writes"]["b"].reshape(1, D),
      lp["rev"]["Wl"], lp["rev"]["Wr"], lp["rev"]["b"].reshape(1, D))
    return oa, op


def kernel(x_author, x_paper, edge_index_writes, edge_index_rev, params):
    xa, xp = x_author, x_paper
    srcw = _pad_edges(edge_index_writes[0])
    dstw = _pad_edges(edge_index_writes[1])
    srcr = _pad_edges(edge_index_rev[0])
    dstr = _pad_edges(edge_index_rev[1])
    zd = jnp.zeros((_NP, D), jnp.float32)
    ones_h = jnp.ones((_CH, D), jnp.float32)
    zrows = jnp.zeros((8, D), jnp.float32)
    cnt = _sc_counts(dstw, dstr, zd, ones_h)
    for i in range(3):
        lp = params["l%d" % i]
        xap = jnp.concatenate([xa, zrows])
        xpp = jnp.concatenate([xp, zrows])
        agg = _sc_agg(xap, xpp, srcw, dstw, srcr, dstr, zd)
        xa, xp = _combine(agg[0], cnt[0], agg[1], cnt[1], xa, xp, lp)
    return (xa, xp)


# speedup vs baseline: 8.4870x; 1.1824x over previous
import functools

import jax
import jax.numpy as jnp
from jax import lax
from jax.experimental import pallas as pl
from jax.experimental.pallas import tpu as pltpu
from jax.experimental.pallas import tpu_sc as plsc

N = 10000
E = 320000
D = 128

_NTILES = 16
_CH = 128
_BLK = 8
_NBLK = 20
_NCHT = _NBLK * _BLK
_EPT = _NCHT * _CH
_EPAD = _NTILES * _EPT
_EROWS = _EPAD // _CH
_NP = N + 8
_RPT = 624
_RPT_LAST_W = N - _RPT * (_NTILES - 1)
_RPT_LAST_Z = _NP - _RPT * (_NTILES - 1)

_BR = 1000


def _row_shard(sid, last, do):
    @pl.when(sid < _NTILES - 1)
    def _():
        do(sid * _RPT, _RPT)

    @pl.when(sid == _NTILES - 1)
    def _():
        do((_NTILES - 1) * _RPT, last)


def _edge_pipe(x_hbm, src_hbm, dst_hbm, row0,
               sblk0, sblk1, dblk0, dblk1, rows0, rows1, acc_sh,
               semS0, semS1, semD0, semD1, semG0, semG1):
    pltpu.async_copy(src_hbm.at[pl.ds(row0, _BLK)], sblk0, semS0)
    pltpu.async_copy(dst_hbm.at[pl.ds(row0, _BLK)], dblk0, semD0)

    def super_step(s, carry):
        r = row0 + 2 * s * _BLK
        pltpu.make_async_copy(src_hbm.at[pl.ds(r, _BLK)], sblk0, semS0).wait()
        pltpu.make_async_copy(dst_hbm.at[pl.ds(r, _BLK)], dblk0, semD0).wait()
        pltpu.async_copy(src_hbm.at[pl.ds(r + _BLK, _BLK)], sblk1, semS1)
        pltpu.async_copy(dst_hbm.at[pl.ds(r + _BLK, _BLK)], dblk1, semD1)
        pltpu.async_copy(x_hbm.at[sblk0.at[0]], rows0, semG0)
        for c in range(2 * _BLK):
            sblk, dblk = (sblk0, dblk0) if c < _BLK else (sblk1, dblk1)
            rw, semw = (rows0, semG0) if c % 2 == 0 else (rows1, semG1)
            rn, semn = (rows1, semG1) if c % 2 == 0 else (rows0, semG0)
            if c == _BLK - 1:
                pltpu.make_async_copy(
                    src_hbm.at[pl.ds(r + _BLK, _BLK)], sblk1, semS1).wait()
                pltpu.make_async_copy(
                    dst_hbm.at[pl.ds(r + _BLK, _BLK)], dblk1, semD1).wait()
            if c < 2 * _BLK - 1:
                nblk = sblk0 if c + 1 < _BLK else sblk1
                pltpu.async_copy(x_hbm.at[nblk.at[(c + 1) % _BLK]], rn, semn)
            if c == 2 * _BLK - 1:
                @pl.when(s < _NBLK // 2 - 1)
                def _():
                    pltpu.async_copy(
                        src_hbm.at[pl.ds(r + 2 * _BLK, _BLK)], sblk0, semS0)
                    pltpu.async_copy(
                        dst_hbm.at[pl.ds(r + 2 * _BLK, _BLK)], dblk0, semD0)
            pltpu.make_async_copy(x_hbm.at[sblk.at[c % _BLK]], rw, semw).wait()
            pltpu.sync_copy(rw, acc_sh.at[dblk.at[c % _BLK]], add=True)
        return carry

    lax.fori_loop(0, _NBLK // 2, super_step, 0)


@functools.partial(
    pl.kernel,
    mesh=plsc.VectorSubcoreMesh(core_axis_name="c", subcore_axis_name="s"),
    out_type=jax.ShapeDtypeStruct((2, N, D), jnp.float32),
    scratch_types=[
        pltpu.VMEM((_BLK, _CH), jnp.int32),
        pltpu.VMEM((_BLK, _CH), jnp.int32),
        pltpu.VMEM((_BLK, _CH), jnp.int32),
        pltpu.VMEM((_BLK, _CH), jnp.int32),
        pltpu.VMEM((_CH, D), jnp.float32),
        pltpu.VMEM((_CH, D), jnp.float32),
        pltpu.VMEM_SHARED((_NP, D), jnp.float32),
        pltpu.SemaphoreType.DMA,
        pltpu.SemaphoreType.DMA,
        pltpu.SemaphoreType.DMA,
        pltpu.SemaphoreType.DMA,
        pltpu.SemaphoreType.DMA,
        pltpu.SemaphoreType.DMA,
    ],
)
def _sc_agg(xa, xp, srcw, dstw, srcr, dstr, zd, out_agg,
            sblk0, sblk1, dblk0, dblk1, rows0, rows1, acc_sh,
            semS0, semS1, semD0, semD1, semG0, semG1):
    cid = lax.axis_index("c")
    sid = lax.axis_index("s")
    _row_shard(sid, _RPT_LAST_Z, lambda r0, nr: pltpu.sync_copy(
        zd.at[pl.ds(r0, nr)], acc_sh.at[pl.ds(r0, nr)]))
    plsc.subcore_barrier()
    row0 = sid * _NCHT

    @pl.when(cid == 0)
    def _():
        _edge_pipe(xa, srcw, dstw, row0, sblk0, sblk1, dblk0, dblk1,
                   rows0, rows1, acc_sh, semS0, semS1, semD0, semD1,
                   semG0, semG1)

    @pl.when(cid == 1)
    def _():
        _edge_pipe(xp, srcr, dstr, row0, sblk0, sblk1, dblk0, dblk1,
                   rows0, rows1, acc_sh, semS0, semS1, semD0, semD1,
                   semG0, semG1)

    plsc.subcore_barrier()
    _row_shard(sid, _RPT_LAST_W, lambda r0, nr: pltpu.sync_copy(
        acc_sh.at[pl.ds(r0, nr)], out_agg.at[cid, pl.ds(r0, nr)]))


@functools.partial(
    pl.kernel,
    mesh=plsc.VectorSubcoreMesh(core_axis_name="c", subcore_axis_name="s"),
    out_type=jax.ShapeDtypeStruct((2, N, D), jnp.float32),
    scratch_types=[
        pltpu.VMEM((_BLK, _CH), jnp.int32),
        pltpu.VMEM((_BLK, _CH), jnp.int32),
        pltpu.VMEM((_CH, D), jnp.float32),
        pltpu.VMEM_SHARED((_NP, D), jnp.float32),
        pltpu.SemaphoreType.DMA,
        pltpu.SemaphoreType.DMA,
    ],
)
def _sc_counts(dstw, dstr, zd, ones_h, out_cnt,
               dblk0, dblk1, ones_v, cnt_sh, semD0, semD1):
    cid = lax.axis_index("c")
    sid = lax.axis_index("s")
    _row_shard(sid, _RPT_LAST_Z, lambda r0, nr: pltpu.sync_copy(
        zd.at[pl.ds(r0, nr)], cnt_sh.at[pl.ds(r0, nr)]))
    pltpu.sync_copy(ones_h, ones_v)
    plsc.subcore_barrier()
    row0 = sid * _NCHT

    def count_loop(dst_hbm):
        pltpu.async_copy(dst_hbm.at[pl.ds(row0, _BLK)], dblk0, semD0)

        def super_step(s, carry):
            r = row0 + 2 * s * _BLK
            pltpu.make_async_copy(
                dst_hbm.at[pl.ds(r, _BLK)], dblk0, semD0).wait()
            pltpu.async_copy(dst_hbm.at[pl.ds(r + _BLK, _BLK)], dblk1, semD1)
            for c in range(_BLK):
                pltpu.sync_copy(ones_v, cnt_sh.at[dblk0.at[c]], add=True)
            pltpu.make_async_copy(
                dst_hbm.at[pl.ds(r + _BLK, _BLK)], dblk1, semD1).wait()

            @pl.when(s < _NBLK // 2 - 1)
            def _():
                pltpu.async_copy(
                    dst_hbm.at[pl.ds(r + 2 * _BLK, _BLK)], dblk0, semD0)

            for c in range(_BLK):
                pltpu.sync_copy(ones_v, cnt_sh.at[dblk1.at[c]], add=True)
            return carry

        lax.fori_loop(0, _NBLK // 2, super_step, 0)

    @pl.when(cid == 0)
    def _():
        count_loop(dstw)

    @pl.when(cid == 1)
    def _():
        count_loop(dstr)

    plsc.subcore_barrier()
    _row_shard(sid, _RPT_LAST_W, lambda r0, nr: pltpu.sync_copy(
        cnt_sh.at[pl.ds(r0, nr)], out_cnt.at[cid, pl.ds(r0, nr)]))


def _pad_edges(a):
    pad = (jnp.arange(_EPAD - E, dtype=jnp.int32) % 8) + N
    return jnp.concatenate([a, pad]).reshape(_EROWS, _CH)


def _mmT(a, b):
    return lax.dot_general(a, b, (((1,), (1,)), ((), ())),
                           preferred_element_type=jnp.float32)


def _combine_body(aggw, cntw, aggr, cntr, xa, xp,
                  wlw, wrw, bw, wlr, wrr, br, oa, op):
    mw = aggw[...] / jnp.maximum(cntw[...][:, 0:1], 1.0)
    mr = aggr[...] / jnp.maximum(cntr[...][:, 0:1], 1.0)
    op[...] = _mmT(mw, wlw[...]) + _mmT(xp[...], wrw[...]) + bw[...]
    oa[...] = _mmT(mr, wlr[...]) + _mmT(xa[...], wrr[...]) + br[...]


def _combine(aggw, cntw, aggr, cntr, xa, xp, lp):
    row_spec = pl.BlockSpec((_BR, D), lambda i: (i, 0))
    w_spec = pl.BlockSpec((D, D), lambda i: (0, 0))
    b_spec = pl.BlockSpec((1, D), lambda i: (0, 0))
    oa, op = pl.pallas_call(
        _combine_body,
        grid=(N // _BR,),
        in_specs=[row_spec, row_spec, row_spec, row_spec, row_spec, row_spec,
                  w_spec, w_spec, b_spec, w_spec, w_spec, b_spec],
        out_specs=[row_spec, row_spec],
        out_shape=[
            jax.ShapeDtypeStruct((N, D), jnp.float32),
            jax.ShapeDtypeStruct((N, D), jnp.float32),
        ],
    )(aggw, cntw, aggr, cntr, xa, xp,
      lp["writes"]["Wl"], lp["writes"]["Wr"], lp["writes"]["b"].reshape(1, D),
      lp["rev"]["Wl"], lp["rev"]["Wr"], lp["rev"]["b"].reshape(1, D))
    return oa, op


def kernel(x_author, x_paper, edge_index_writes, edge_index_rev, params):
    xa, xp = x_author, x_paper
    srcw = _pad_edges(edge_index_writes[0])
    dstw = _pad_edges(edge_index_writes[1])
    srcr = _pad_edges(edge_index_rev[0])
    dstr = _pad_edges(edge_index_rev[1])
    zd = jnp.zeros((_NP, D), jnp.float32)
    ones_h = jnp.ones((_CH, D), jnp.float32)
    zrows = jnp.zeros((8, D), jnp.float32)
    cnt = _sc_counts(dstw, dstr, zd, ones_h)
    for i in range(3):
        lp = params["l%d" % i]
        xap = jnp.concatenate([xa, zrows])
        xpp = jnp.concatenate([xp, zrows])
        agg = _sc_agg(xap, xpp, srcw, dstw, srcr, dstr, zd)
        xa, xp = _combine(agg[0], cnt[0], agg[1], cnt[1], xa, xp, lp)
    return (xa, xp)


# speedup vs baseline: 8.5204x; 1.0039x over previous
import functools

import jax
import jax.numpy as jnp
from jax import lax
from jax.experimental import pallas as pl
from jax.experimental.pallas import tpu as pltpu
from jax.experimental.pallas import tpu_sc as plsc

N = 10000
E = 320000
D = 128

_NTILES = 16
_CH = 128
_BLK = 8
_NBLK = 20
_NCHT = _NBLK * _BLK
_EPT = _NCHT * _CH
_EPAD = _NTILES * _EPT
_EROWS = _EPAD // _CH
_NP = N + 8
_RPT = 624
_RPT_LAST_W = N - _RPT * (_NTILES - 1)
_RPT_LAST_Z = _NP - _RPT * (_NTILES - 1)

_BR = 1000


def _row_shard(sid, last, do):
    @pl.when(sid < _NTILES - 1)
    def _():
        do(sid * _RPT, _RPT)

    @pl.when(sid == _NTILES - 1)
    def _():
        do((_NTILES - 1) * _RPT, last)


def _edge_pipe(x_hbm, src_hbm, dst_hbm, row0,
               sblk0, sblk1, dblk0, dblk1, rows0, rows1, acc_sh,
               semS0, semS1, semD0, semD1, semG0, semG1):
    pltpu.async_copy(src_hbm.at[pl.ds(row0, _BLK)], sblk0, semS0)
    pltpu.async_copy(dst_hbm.at[pl.ds(row0, _BLK)], dblk0, semD0)

    def super_step(s, carry):
        r = row0 + 2 * s * _BLK
        pltpu.make_async_copy(src_hbm.at[pl.ds(r, _BLK)], sblk0, semS0).wait()
        pltpu.make_async_copy(dst_hbm.at[pl.ds(r, _BLK)], dblk0, semD0).wait()
        pltpu.async_copy(src_hbm.at[pl.ds(r + _BLK, _BLK)], sblk1, semS1)
        pltpu.async_copy(dst_hbm.at[pl.ds(r + _BLK, _BLK)], dblk1, semD1)
        pltpu.async_copy(x_hbm.at[sblk0.at[0]], rows0, semG0)
        for c in range(2 * _BLK):
            sblk, dblk = (sblk0, dblk0) if c < _BLK else (sblk1, dblk1)
            rw, semw = (rows0, semG0) if c % 2 == 0 else (rows1, semG1)
            rn, semn = (rows1, semG1) if c % 2 == 0 else (rows0, semG0)
            if c == _BLK - 1:
                pltpu.make_async_copy(
                    src_hbm.at[pl.ds(r + _BLK, _BLK)], sblk1, semS1).wait()
                pltpu.make_async_copy(
                    dst_hbm.at[pl.ds(r + _BLK, _BLK)], dblk1, semD1).wait()
            if c < 2 * _BLK - 1:
                nblk = sblk0 if c + 1 < _BLK else sblk1
                pltpu.async_copy(x_hbm.at[nblk.at[(c + 1) % _BLK]], rn, semn)
            if c == 2 * _BLK - 1:
                @pl.when(s < _NBLK // 2 - 1)
                def _():
                    pltpu.async_copy(
                        src_hbm.at[pl.ds(r + 2 * _BLK, _BLK)], sblk0, semS0)
                    pltpu.async_copy(
                        dst_hbm.at[pl.ds(r + 2 * _BLK, _BLK)], dblk0, semD0)
            pltpu.make_async_copy(x_hbm.at[sblk.at[c % _BLK]], rw, semw).wait()
            pltpu.sync_copy(rw, acc_sh.at[dblk.at[c % _BLK]], add=True)
        return carry

    lax.fori_loop(0, _NBLK // 2, super_step, 0)


@functools.partial(
    pl.kernel,
    mesh=plsc.VectorSubcoreMesh(core_axis_name="c", subcore_axis_name="s"),
    out_type=jax.ShapeDtypeStruct((2, N, D), jnp.float32),
    scratch_types=[
        pltpu.VMEM((_BLK, _CH), jnp.int32),
        pltpu.VMEM((_BLK, _CH), jnp.int32),
        pltpu.VMEM((_BLK, _CH), jnp.int32),
        pltpu.VMEM((_BLK, _CH), jnp.int32),
        pltpu.VMEM((_CH, D), jnp.float32),
        pltpu.VMEM((_CH, D), jnp.float32),
        pltpu.VMEM_SHARED((_NP, D), jnp.float32),
        pltpu.SemaphoreType.DMA,
        pltpu.SemaphoreType.DMA,
        pltpu.SemaphoreType.DMA,
        pltpu.SemaphoreType.DMA,
        pltpu.SemaphoreType.DMA,
        pltpu.SemaphoreType.DMA,
    ],
)
def _sc_agg(xa, xp, srcw, dstw, srcr, dstr, zd, out_agg,
            sblk0, sblk1, dblk0, dblk1, rows0, rows1, acc_sh,
            semS0, semS1, semD0, semD1, semG0, semG1):
    cid = lax.axis_index("c")
    sid = lax.axis_index("s")
    _row_shard(sid, _RPT_LAST_Z, lambda r0, nr: pltpu.sync_copy(
        zd.at[pl.ds(r0, nr)], acc_sh.at[pl.ds(r0, nr)]))
    plsc.subcore_barrier()
    row0 = sid * _NCHT

    @pl.when(cid == 0)
    def _():
        _edge_pipe(xa, srcw, dstw, row0, sblk0, sblk1, dblk0, dblk1,
                   rows0, rows1, acc_sh, semS0, semS1, semD0, semD1,
                   semG0, semG1)

    @pl.when(cid == 1)
    def _():
        _edge_pipe(xp, srcr, dstr, row0, sblk0, sblk1, dblk0, dblk1,
                   rows0, rows1, acc_sh, semS0, semS1, semD0, semD1,
                   semG0, semG1)

    plsc.subcore_barrier()
    _row_shard(sid, _RPT_LAST_W, lambda r0, nr: pltpu.sync_copy(
        acc_sh.at[pl.ds(r0, nr)], out_agg.at[cid, pl.ds(r0, nr)]))


@functools.partial(
    pl.kernel,
    mesh=plsc.VectorSubcoreMesh(core_axis_name="c", subcore_axis_name="s"),
    out_type=jax.ShapeDtypeStruct((2, N, D), jnp.float32),
    scratch_types=[
        pltpu.VMEM((_BLK, _CH), jnp.int32),
        pltpu.VMEM((_BLK, _CH), jnp.int32),
        pltpu.VMEM((_CH, D), jnp.float32),
        pltpu.VMEM_SHARED((_NP, D), jnp.float32),
        pltpu.SemaphoreType.DMA,
        pltpu.SemaphoreType.DMA,
    ],
)
def _sc_counts(dstw, dstr, zd, ones_h, out_cnt,
               dblk0, dblk1, ones_v, cnt_sh, semD0, semD1):
    cid = lax.axis_index("c")
    sid = lax.axis_index("s")
    _row_shard(sid, _RPT_LAST_Z, lambda r0, nr: pltpu.sync_copy(
        zd.at[pl.ds(r0, nr)], cnt_sh.at[pl.ds(r0, nr)]))
    pltpu.sync_copy(ones_h, ones_v)
    plsc.subcore_barrier()
    row0 = sid * _NCHT

    def count_loop(dst_hbm):
        pltpu.async_copy(dst_hbm.at[pl.ds(row0, _BLK)], dblk0, semD0)

        def super_step(s, carry):
            r = row0 + 2 * s * _BLK
            pltpu.make_async_copy(
                dst_hbm.at[pl.ds(r, _BLK)], dblk0, semD0).wait()
            pltpu.async_copy(dst_hbm.at[pl.ds(r + _BLK, _BLK)], dblk1, semD1)
            for c in range(_BLK):
                pltpu.sync_copy(ones_v, cnt_sh.at[dblk0.at[c]], add=True)
            pltpu.make_async_copy(
                dst_hbm.at[pl.ds(r + _BLK, _BLK)], dblk1, semD1).wait()

            @pl.when(s < _NBLK // 2 - 1)
            def _():
                pltpu.async_copy(
                    dst_hbm.at[pl.ds(r + 2 * _BLK, _BLK)], dblk0, semD0)

            for c in range(_BLK):
                pltpu.sync_copy(ones_v, cnt_sh.at[dblk1.at[c]], add=True)
            return carry

        lax.fori_loop(0, _NBLK // 2, super_step, 0)

    @pl.when(cid == 0)
    def _():
        count_loop(dstw)

    @pl.when(cid == 1)
    def _():
        count_loop(dstr)

    plsc.subcore_barrier()
    _row_shard(sid, _RPT_LAST_W, lambda r0, nr: pltpu.sync_copy(
        cnt_sh.at[pl.ds(r0, nr)], out_cnt.at[cid, pl.ds(r0, nr)]))


def _pad_edges(a):
    pad = (jnp.arange(_EPAD - E, dtype=jnp.int32) % 8) + N
    return jnp.concatenate([a, pad]).reshape(_EROWS, _CH)


def _mmT(a, b):
    return lax.dot_general(a, b, (((1,), (1,)), ((), ())),
                           preferred_element_type=jnp.float32)


def _combine_body(aggw, cntw, aggr, cntr, xa, xp,
                  wlw, wrw, bw, wlr, wrr, br, oa, op):
    mw = aggw[...] / jnp.maximum(cntw[...][:, 0:1], 1.0)
    mr = aggr[...] / jnp.maximum(cntr[...][:, 0:1], 1.0)
    op[...] = _mmT(mw, wlw[...]) + _mmT(xp[...], wrw[...]) + bw[...]
    oa[...] = _mmT(mr, wlr[...]) + _mmT(xa[...], wrr[...]) + br[...]


def _combine(aggw, cntw, aggr, cntr, xa, xp, lp):
    row_spec = pl.BlockSpec((_BR, D), lambda i: (i, 0))
    w_spec = pl.BlockSpec((D, D), lambda i: (0, 0))
    b_spec = pl.BlockSpec((1, D), lambda i: (0, 0))
    oa, op = pl.pallas_call(
        _combine_body,
        grid=(N // _BR,),
        in_specs=[row_spec, row_spec, row_spec, row_spec, row_spec, row_spec,
                  w_spec, w_spec, b_spec, w_spec, w_spec, b_spec],
        out_specs=[row_spec, row_spec],
        out_shape=[
            jax.ShapeDtypeStruct((_NP, D), jnp.float32),
            jax.ShapeDtypeStruct((_NP, D), jnp.float32),
        ],
    )(aggw, cntw, aggr, cntr, xa, xp,
      lp["writes"]["Wl"], lp["writes"]["Wr"], lp["writes"]["b"].reshape(1, D),
      lp["rev"]["Wl"], lp["rev"]["Wr"], lp["rev"]["b"].reshape(1, D))
    return oa, op


def kernel(x_author, x_paper, edge_index_writes, edge_index_rev, params):
    srcw = _pad_edges(edge_index_writes[0])
    dstw = _pad_edges(edge_index_writes[1])
    srcr = _pad_edges(edge_index_rev[0])
    dstr = _pad_edges(edge_index_rev[1])
    zd = jnp.zeros((_NP, D), jnp.float32)
    ones_h = jnp.ones((_CH, D), jnp.float32)
    zrows = jnp.zeros((8, D), jnp.float32)
    xa = jnp.concatenate([x_author, zrows])
    xp = jnp.concatenate([x_paper, zrows])
    cnt = _sc_counts(dstw, dstr, zd, ones_h)
    for i in range(3):
        lp = params["l%d" % i]
        agg = _sc_agg(xa, xp, srcw, dstw, srcr, dstr, zd)
        xa, xp = _combine(agg[0], cnt[0], agg[1], cnt[1], xa, xp, lp)
    return (xa[:N], xp[:N])
